# Initial kernel scaffold; baseline (speedup 1.0000x reference)
import jax
import jax.numpy as jnp
from jax.experimental import pallas as pl


def kernel(hidden_states, Wq, Wk, Wv, Wo, hash_proj, hash_bias, ln1_w, ln2_w, Wgate, Wup, Wdown):
    raise NotImplementedError("write your pallas kernel here")


# speedup vs baseline: 33.8666x; 33.8666x over previous
import functools
import math

import jax
import jax.numpy as jnp
from jax import lax
from jax.experimental import pallas as pl
from jax.experimental.pallas import tpu as pltpu

F32 = jnp.float32
BF16 = jnp.bfloat16
HI = lax.Precision.HIGHEST


def _dot(a, b, precision=None, trans_b=False):
    dn = (((1,), (1 if trans_b else 0,)), ((), ()))
    return lax.dot_general(a, b, dn, precision=precision,
                           preferred_element_type=F32)


def _rms_body(x_ref, w_ref, o_ref, *, odtype):
    x = x_ref[...]
    var = jnp.mean(x * x, axis=1, keepdims=True)
    o_ref[...] = (x * lax.rsqrt(var + 1e-6) * w_ref[...]).astype(odtype)


def _rmsnorm_p(x, w, odtype=F32):
    S, D = x.shape
    mb = min(256, S)
    return pl.pallas_call(
        functools.partial(_rms_body, odtype=odtype),
        grid=(S // mb,),
        in_specs=[pl.BlockSpec((mb, D), lambda i: (i, 0)),
                  pl.BlockSpec((1, D), lambda i: (0, 0))],
        out_specs=pl.BlockSpec((mb, D), lambda i: (i, 0)),
        out_shape=jax.ShapeDtypeStruct((S, D), odtype),
    )(x, w.reshape(1, D))


def _mm_body(x_ref, w_ref, o_ref, *, prec):
    o_ref[...] = _dot(x_ref[...], w_ref[...], precision=prec)


def _mmres_body(x_ref, w_ref, r_ref, o_ref, *, prec):
    o_ref[...] = _dot(x_ref[...], w_ref[...], precision=prec) + r_ref[...]


def _matmul_p(x, w, mb, nb, prec=None, resid=None):
    M, K = x.shape
    _, N = w.shape
    mb, nb = min(mb, M), min(nb, N)
    grid = (N // nb, M // mb)
    in_specs = [pl.BlockSpec((mb, K), lambda n, m: (m, 0)),
                pl.BlockSpec((K, nb), lambda n, m: (0, n))]
    args = [x, w]
    if resid is not None:
        in_specs.append(pl.BlockSpec((mb, nb), lambda n, m: (m, n)))
        args.append(resid)
        body = functools.partial(_mmres_body, prec=prec)
    else:
        body = functools.partial(_mm_body, prec=prec)
    return pl.pallas_call(
        body, grid=grid,
        in_specs=in_specs,
        out_specs=pl.BlockSpec((mb, nb), lambda n, m: (m, n)),
        out_shape=jax.ShapeDtypeStruct((M, N), F32),
    )(*args)


def _ropehash_body(q_ref, k_ref, cos_ref, sin_ref, p_ref, b_ref,
                   qr_ref, kr_ref, qh_ref, kh_ref):
    cos = cos_ref[...]
    sin = sin_ref[...]
    p = p_ref[0]
    b = b_ref[0]
    hd = cos.shape[1]

    def rope(x):
        x1 = x[:, :hd // 2]
        x2 = x[:, hd // 2:]
        rot = jnp.concatenate([-x2, x1], axis=1)
        return x * cos + rot * sin

    qr = rope(q_ref[...]).astype(BF16)
    kr = rope(k_ref[...]).astype(BF16)
    qr_ref[...] = qr
    kr_ref[...] = kr
    pb = p.astype(BF16)
    qh_ref[...] = jnp.sign(_dot(qr, pb) + b).astype(BF16)
    kh_ref[...] = jnp.sign(_dot(kr, pb) + b).astype(BF16)


def _ropehash_p(qkv, cos, sin, proj, bias, S, D, H, KVH, HD):
    rep = H // KVH
    out_sd = jax.ShapeDtypeStruct((S, H * HD), BF16)
    return pl.pallas_call(
        _ropehash_body,
        grid=(H,),
        in_specs=[
            pl.BlockSpec((S, HD), lambda h: (0, h)),
            pl.BlockSpec((S, HD), lambda h: (0, H + h // rep)),
            pl.BlockSpec((S, HD), lambda h: (0, 0)),
            pl.BlockSpec((S, HD), lambda h: (0, 0)),
            pl.BlockSpec((1, HD, HD), lambda h: (h, 0, 0)),
            pl.BlockSpec((1, 1, HD), lambda h: (h, 0, 0)),
        ],
        out_specs=[pl.BlockSpec((S, HD), lambda h: (0, h)) for _ in range(4)],
        out_shape=[out_sd, out_sd, out_sd, out_sd],
    )(qkv, qkv, cos, sin, proj, bias)


def _attn_body(qh_ref, kh_ref, qr_ref, kr_ref, v_ref, u_ref, o_ref, *,
               qb, s, hd, k_keep, iters, ch):
    draft = _dot(qh_ref[...], kh_ref[...], trans_b=True)
    q0 = pl.program_id(1) * qb
    row = q0 + lax.broadcasted_iota(jnp.int32, (qb, s), 0)
    col = lax.broadcasted_iota(jnp.int32, (qb, s), 1)
    inband = col <= row
    draftm = jnp.where(inband, draft, -1000.0)

    kf = jnp.float32(k_keep)
    lo = jnp.full((qb, 1), -(hd + 1.0), F32)
    hi = jnp.full((qb, 1), hd + 1.0, F32)
    for _ in range(iters):
        mid = jnp.floor((lo + hi) * 0.5)
        cnt = jnp.sum((draftm >= mid).astype(F32), axis=1, keepdims=True)
        ge = cnt >= kf
        lo = jnp.where(ge, mid, lo)
        hi = jnp.where(ge, hi, mid)
    t = lo
    c_gt = jnp.sum((draftm > t).astype(F32), axis=1, keepdims=True)
    need = kf - c_gt

    tie = (draftm == t).astype(BF16)
    u = u_ref[...]
    ranks = []
    offset = jnp.zeros((qb, 1), F32)
    for c in range(s // ch):
        chunk = lax.slice(tie, (0, c * ch), (qb, (c + 1) * ch))
        wc = _dot(chunk, u)
        ranks.append(wc + offset)
        offset = offset + lax.slice(wc, (0, ch - 1), (qb, ch))
    rank = jnp.concatenate(ranks, axis=1)
    keep = (draftm > t) | ((draftm == t) & (rank <= need))

    sc = _dot(qr_ref[...], kr_ref[...], trans_b=True) * (1.0 / math.sqrt(hd))
    sc = jnp.where(keep, sc, -1e30)
    m = jnp.max(sc, axis=1, keepdims=True)
    p = jnp.exp(sc - m)
    probs = (p / jnp.sum(p, axis=1, keepdims=True)).astype(BF16)
    o_ref[...] = _dot(probs, v_ref[...].astype(BF16)).astype(BF16)


def _attn_p(qh, kh, qr, kr, qkv, tri, S, D, H, KVH, HD, k_keep):
    qb = min(256, S)
    ch = min(256, S)
    iters = max(1, math.ceil(math.log2(2 * hd_range(HD))))
    rep = H // KVH
    body = functools.partial(_attn_body, qb=qb, s=S, hd=HD, k_keep=k_keep,
                             iters=iters, ch=ch)
    return pl.pallas_call(
        body,
        grid=(H, S // qb),
        in_specs=[
            pl.BlockSpec((qb, HD), lambda h, q: (q, h)),
            pl.BlockSpec((S, HD), lambda h, q: (0, h)),
            pl.BlockSpec((qb, HD), lambda h, q: (q, h)),
            pl.BlockSpec((S, HD), lambda h, q: (0, h)),
            pl.BlockSpec((S, HD), lambda h, q: (0, H + KVH + h // rep)),
            pl.BlockSpec((ch, ch), lambda h, q: (0, 0)),
        ],
        out_specs=pl.BlockSpec((qb, HD), lambda h, q: (q, h)),
        out_shape=jax.ShapeDtypeStruct((S, H * HD), BF16),
    )(qh, kh, qr, kr, qkv, tri)


def hd_range(HD):
    return HD + 1


def _mlp_body(h2_ref, wg_ref, wu_ref, wd_ref, o_ref):
    g = _dot(h2_ref[...], wg_ref[...])
    uu = _dot(h2_ref[...], wu_ref[...])
    act = (g * lax.logistic(g) * uu).astype(BF16)
    pd = _dot(act, wd_ref[...])

    @pl.when(pl.program_id(1) == 0)
    def _():
        o_ref[...] = pd

    @pl.when(pl.program_id(1) != 0)
    def _():
        o_ref[...] += pd


def _mlp_p(h2, wg, wu, wd):
    S, D = h2.shape
    FF = wg.shape[1]
    sb = min(512, S)
    fb = min(256, FF)
    return pl.pallas_call(
        _mlp_body,
        grid=(S // sb, FF // fb),
        in_specs=[
            pl.BlockSpec((sb, D), lambda s, f: (s, 0)),
            pl.BlockSpec((D, fb), lambda s, f: (0, f)),
            pl.BlockSpec((D, fb), lambda s, f: (0, f)),
            pl.BlockSpec((fb, D), lambda s, f: (f, 0)),
        ],
        out_specs=pl.BlockSpec((sb, D), lambda s, f: (s, 0)),
        out_shape=jax.ShapeDtypeStruct((S, D), F32),
    )(h2, wg, wu, wd)


def kernel(hidden_states, Wq, Wk, Wv, Wo, hash_proj, hash_bias,
           ln1_w, ln2_w, Wgate, Wup, Wdown):
    B, S, D = hidden_states.shape
    HHD = Wq.shape[1]
    HD = hash_proj.shape[-1]
    H = HHD // HD
    KVH = Wk.shape[1] // HD
    theta = 10000.0
    num_remain = S - int(S * 0.9)
    num_remain = max(min(S, 128), num_remain)

    hs2 = hidden_states.reshape(S, D)

    inv_freq = 1.0 / (theta ** (jnp.arange(0, HD, 2, dtype=F32) / HD))
    tpos = jnp.arange(S, dtype=F32)
    freqs = jnp.outer(tpos, inv_freq)
    emb = jnp.concatenate([freqs, freqs], axis=-1)
    cos, sin = jnp.cos(emb), jnp.sin(emb)

    tri = (lax.broadcasted_iota(jnp.int32, (min(256, S),) * 2, 0)
           <= lax.broadcasted_iota(jnp.int32, (min(256, S),) * 2, 1)
           ).astype(BF16)

    h1 = _rmsnorm_p(hs2, ln1_w, odtype=BF16)
    wqkv = jnp.concatenate([Wq, Wk, Wv], axis=1).astype(BF16)
    qkv = _matmul_p(h1, wqkv, mb=512, nb=512)

    proj = hash_proj.reshape(H, HD, HD)
    bias = hash_bias.reshape(H, 1, HD)
    qr, kr, qh, kh = _ropehash_p(qkv, cos, sin, proj, bias, S, D, H, KVH, HD)

    attn = _attn_p(qh, kh, qr, kr, qkv, tri, S, D, H, KVH, HD, num_remain)

    resid2 = _matmul_p(attn, Wo.astype(BF16), mb=512, nb=1024, resid=hs2)

    h2 = _rmsnorm_p(resid2, ln2_w, odtype=BF16)
    mlp = _mlp_p(h2, Wgate.astype(BF16), Wup.astype(BF16), Wdown.astype(BF16))
    return (resid2 + mlp).reshape(B, S, D)


# speedup vs baseline: 39.3383x; 1.1616x over previous
import functools
import math

import jax
import jax.numpy as jnp
from jax import lax
from jax.experimental import pallas as pl
from jax.experimental.pallas import tpu as pltpu

F32 = jnp.float32
BF16 = jnp.bfloat16
HI = lax.Precision.HIGHEST


def _dot(a, b, precision=None, trans_b=False):
    dn = (((1,), (1 if trans_b else 0,)), ((), ()))
    return lax.dot_general(a, b, dn, precision=precision,
                           preferred_element_type=F32)


def _rms_body(x_ref, w_ref, o_ref, *, odtype):
    x = x_ref[...]
    var = jnp.mean(x * x, axis=1, keepdims=True)
    o_ref[...] = (x * lax.rsqrt(var + 1e-6) * w_ref[...]).astype(odtype)


def _rmsnorm_p(x, w, odtype=F32):
    S, D = x.shape
    mb = min(256, S)
    return pl.pallas_call(
        functools.partial(_rms_body, odtype=odtype),
        grid=(S // mb,),
        in_specs=[pl.BlockSpec((mb, D), lambda i: (i, 0)),
                  pl.BlockSpec((1, D), lambda i: (0, 0))],
        out_specs=pl.BlockSpec((mb, D), lambda i: (i, 0)),
        out_shape=jax.ShapeDtypeStruct((S, D), odtype),
    )(x, w.reshape(1, D))


def _mm_body(x_ref, w_ref, o_ref, *, prec):
    o_ref[...] = _dot(x_ref[...], w_ref[...], precision=prec)


def _mmres_body(x_ref, w_ref, r_ref, o_ref, *, prec):
    o_ref[...] = _dot(x_ref[...], w_ref[...], precision=prec) + r_ref[...]


def _matmul_p(x, w, mb, nb, prec=None, resid=None):
    M, K = x.shape
    _, N = w.shape
    mb, nb = min(mb, M), min(nb, N)
    grid = (N // nb, M // mb)
    in_specs = [pl.BlockSpec((mb, K), lambda n, m: (m, 0)),
                pl.BlockSpec((K, nb), lambda n, m: (0, n))]
    args = [x, w]
    if resid is not None:
        in_specs.append(pl.BlockSpec((mb, nb), lambda n, m: (m, n)))
        args.append(resid)
        body = functools.partial(_mmres_body, prec=prec)
    else:
        body = functools.partial(_mm_body, prec=prec)
    return pl.pallas_call(
        body, grid=grid,
        in_specs=in_specs,
        out_specs=pl.BlockSpec((mb, nb), lambda n, m: (m, n)),
        out_shape=jax.ShapeDtypeStruct((M, N), F32),
    )(*args)


def _ropehash_body(q_ref, k_ref, cos_ref, sin_ref, p_ref, b_ref,
                   qr_ref, kr_ref, qh_ref, kh_ref):
    cos = cos_ref[...]
    sin = sin_ref[...]
    p = p_ref[0]
    b = b_ref[0]
    hd = cos.shape[1]

    def rope(x):
        x1 = x[:, :hd // 2]
        x2 = x[:, hd // 2:]
        rot = jnp.concatenate([-x2, x1], axis=1)
        return x * cos + rot * sin

    qr = rope(q_ref[...]).astype(BF16)
    kr = rope(k_ref[...]).astype(BF16)
    qr_ref[...] = qr
    kr_ref[...] = kr
    pb = p.astype(BF16)
    qh_ref[...] = jnp.sign(_dot(qr, pb) + b).astype(BF16)
    kh_ref[...] = jnp.sign(_dot(kr, pb) + b).astype(BF16)


def _ropehash_p(qkv, cos, sin, proj, bias, S, D, H, KVH, HD):
    rep = H // KVH
    out_sd = jax.ShapeDtypeStruct((S, H * HD), BF16)
    return pl.pallas_call(
        _ropehash_body,
        grid=(H,),
        in_specs=[
            pl.BlockSpec((S, HD), lambda h: (0, h)),
            pl.BlockSpec((S, HD), lambda h: (0, H + h // rep)),
            pl.BlockSpec((S, HD), lambda h: (0, 0)),
            pl.BlockSpec((S, HD), lambda h: (0, 0)),
            pl.BlockSpec((1, HD, HD), lambda h: (h, 0, 0)),
            pl.BlockSpec((1, 1, HD), lambda h: (h, 0, 0)),
        ],
        out_specs=[pl.BlockSpec((S, HD), lambda h: (0, h)) for _ in range(4)],
        out_shape=[out_sd, out_sd, out_sd, out_sd],
    )(qkv, qkv, cos, sin, proj, bias)


def _attn_body(qh_ref, kh_ref, qr_ref, kr_ref, v_ref, u_ref, o_ref, *,
               qb, s, hd, k_keep, iters, ch):
    draft = _dot(qh_ref[...], kh_ref[...], trans_b=True)
    q0 = pl.program_id(1) * qb
    row = q0 + lax.broadcasted_iota(jnp.int32, (qb, s), 0)
    col = lax.broadcasted_iota(jnp.int32, (qb, s), 1)
    inband = col <= row
    draftm = jnp.where(inband, draft, -1000.0)

    kf = jnp.float32(k_keep)
    lo = jnp.full((qb, 1), -(hd + 1.0), F32)
    hi = jnp.full((qb, 1), hd + 1.0, F32)
    cnt_hi = jnp.zeros((qb, 1), F32)
    for _ in range(iters):
        mid = jnp.floor((lo + hi) * 0.5)
        cnt = jnp.sum((draftm >= mid).astype(F32), axis=1, keepdims=True)
        ge = cnt >= kf
        lo = jnp.where(ge, mid, lo)
        hi = jnp.where(ge, hi, mid)
        cnt_hi = jnp.where(ge, cnt_hi, cnt)
    t = lo
    need = kf - cnt_hi

    tie = (draftm == t).astype(BF16)
    u = u_ref[...]
    ranks = []
    offset = jnp.zeros((qb, 1), F32)
    for c in range(s // ch):
        chunk = lax.slice(tie, (0, c * ch), (qb, (c + 1) * ch))
        wc = _dot(chunk, u)
        ranks.append(wc + offset)
        offset = offset + lax.slice(wc, (0, ch - 1), (qb, ch))
    rank = jnp.concatenate(ranks, axis=1)
    keep = (draftm > t) | ((draftm == t) & (rank <= need))

    sc = _dot(qr_ref[...], kr_ref[...], trans_b=True) * (1.0 / math.sqrt(hd))
    sc = jnp.where(keep, sc, -1e30)
    m = jnp.max(sc, axis=1, keepdims=True)
    p = jnp.exp(sc - m)
    probs = (p / jnp.sum(p, axis=1, keepdims=True)).astype(BF16)
    o_ref[...] = _dot(probs, v_ref[...].astype(BF16)).astype(BF16)


def _attn_p(qh, kh, qr, kr, qkv, tri, S, D, H, KVH, HD, k_keep):
    qb = min(512, S)
    ch = min(256, S)
    iters = max(1, math.ceil(math.log2(2 * hd_range(HD))))
    rep = H // KVH
    body = functools.partial(_attn_body, qb=qb, s=S, hd=HD, k_keep=k_keep,
                             iters=iters, ch=ch)
    return pl.pallas_call(
        body,
        grid=(H, S // qb),
        in_specs=[
            pl.BlockSpec((qb, HD), lambda h, q: (q, h)),
            pl.BlockSpec((S, HD), lambda h, q: (0, h)),
            pl.BlockSpec((qb, HD), lambda h, q: (q, h)),
            pl.BlockSpec((S, HD), lambda h, q: (0, h)),
            pl.BlockSpec((S, HD), lambda h, q: (0, H + KVH + h // rep)),
            pl.BlockSpec((ch, ch), lambda h, q: (0, 0)),
        ],
        out_specs=pl.BlockSpec((qb, HD), lambda h, q: (q, h)),
        out_shape=jax.ShapeDtypeStruct((S, H * HD), BF16),
    )(qh, kh, qr, kr, qkv, tri)


def hd_range(HD):
    return HD + 1


def _gate_up_body(h2_ref, wg_ref, wu_ref, act_ref):
    h2 = h2_ref[...]
    g = _dot(h2, wg_ref[...].astype(BF16))
    uu = _dot(h2, wu_ref[...].astype(BF16))
    act_ref[...] = (g * lax.logistic(g) * uu).astype(BF16)


def _gate_up_p(h2, wg, wu):
    S, D = h2.shape
    FF = wg.shape[1]
    fb = min(256, FF)
    return pl.pallas_call(
        _gate_up_body,
        grid=(FF // fb,),
        in_specs=[
            pl.BlockSpec((S, D), lambda f: (0, 0)),
            pl.BlockSpec((D, fb), lambda f: (0, f)),
            pl.BlockSpec((D, fb), lambda f: (0, f)),
        ],
        out_specs=pl.BlockSpec((S, fb), lambda f: (0, f)),
        out_shape=jax.ShapeDtypeStruct((S, FF), BF16),
    )(h2, wg, wu)


def kernel(hidden_states, Wq, Wk, Wv, Wo, hash_proj, hash_bias,
           ln1_w, ln2_w, Wgate, Wup, Wdown):
    B, S, D = hidden_states.shape
    HHD = Wq.shape[1]
    HD = hash_proj.shape[-1]
    H = HHD // HD
    KVH = Wk.shape[1] // HD
    theta = 10000.0
    num_remain = S - int(S * 0.9)
    num_remain = max(min(S, 128), num_remain)

    hs2 = hidden_states.reshape(S, D)

    inv_freq = 1.0 / (theta ** (jnp.arange(0, HD, 2, dtype=F32) / HD))
    tpos = jnp.arange(S, dtype=F32)
    freqs = jnp.outer(tpos, inv_freq)
    emb = jnp.concatenate([freqs, freqs], axis=-1)
    cos, sin = jnp.cos(emb), jnp.sin(emb)

    tri = (lax.broadcasted_iota(jnp.int32, (min(256, S),) * 2, 0)
           <= lax.broadcasted_iota(jnp.int32, (min(256, S),) * 2, 1)
           ).astype(BF16)

    h1 = _rmsnorm_p(hs2, ln1_w, odtype=BF16)
    wqkv = jnp.concatenate([Wq, Wk, Wv], axis=1).astype(BF16)
    qkv = _matmul_p(h1, wqkv, mb=512, nb=1024)

    proj = hash_proj.reshape(H, HD, HD)
    bias = hash_bias.reshape(H, 1, HD)
    qr, kr, qh, kh = _ropehash_p(qkv, cos, sin, proj, bias, S, D, H, KVH, HD)

    attn = _attn_p(qh, kh, qr, kr, qkv, tri, S, D, H, KVH, HD, num_remain)

    resid2 = _matmul_p(attn, Wo.astype(BF16), mb=512, nb=1024, resid=hs2)

    h2 = _rmsnorm_p(resid2, ln2_w, odtype=BF16)
    act = _gate_up_p(h2, Wgate, Wup)
    mlp = _matmul_p(act, Wdown.astype(BF16), mb=512, nb=512)
    return (resid2 + mlp).reshape(B, S, D)


# speedup vs baseline: 40.9613x; 1.0413x over previous
import functools
import math

import jax
import jax.numpy as jnp
from jax import lax
from jax.experimental import pallas as pl
from jax.experimental.pallas import tpu as pltpu

F32 = jnp.float32
BF16 = jnp.bfloat16
HI = lax.Precision.HIGHEST


def _dot(a, b, precision=None, trans_b=False):
    dn = (((1,), (1 if trans_b else 0,)), ((), ()))
    return lax.dot_general(a, b, dn, precision=precision,
                           preferred_element_type=F32)


def _rms_body(x_ref, w_ref, o_ref, *, odtype):
    x = x_ref[...]
    var = jnp.mean(x * x, axis=1, keepdims=True)
    o_ref[...] = (x * lax.rsqrt(var + 1e-6) * w_ref[...]).astype(odtype)


def _rmsnorm_p(x, w, odtype=F32):
    S, D = x.shape
    mb = min(256, S)
    return pl.pallas_call(
        functools.partial(_rms_body, odtype=odtype),
        grid=(S // mb,),
        in_specs=[pl.BlockSpec((mb, D), lambda i: (i, 0)),
                  pl.BlockSpec((1, D), lambda i: (0, 0))],
        out_specs=pl.BlockSpec((mb, D), lambda i: (i, 0)),
        out_shape=jax.ShapeDtypeStruct((S, D), odtype),
    )(x, w.reshape(1, D))


def _mm_body(x_ref, w_ref, o_ref, *, prec):
    o_ref[...] = _dot(x_ref[...], w_ref[...], precision=prec)


def _mmres_body(x_ref, w_ref, r_ref, o_ref, *, prec):
    o_ref[...] = _dot(x_ref[...], w_ref[...], precision=prec) + r_ref[...]


def _matmul_p(x, w, mb, nb, prec=None, resid=None):
    M, K = x.shape
    _, N = w.shape
    mb, nb = min(mb, M), min(nb, N)
    grid = (N // nb, M // mb)
    in_specs = [pl.BlockSpec((mb, K), lambda n, m: (m, 0)),
                pl.BlockSpec((K, nb), lambda n, m: (0, n))]
    args = [x, w]
    if resid is not None:
        in_specs.append(pl.BlockSpec((mb, nb), lambda n, m: (m, n)))
        args.append(resid)
        body = functools.partial(_mmres_body, prec=prec)
    else:
        body = functools.partial(_mm_body, prec=prec)
    return pl.pallas_call(
        body, grid=grid,
        in_specs=in_specs,
        out_specs=pl.BlockSpec((mb, nb), lambda n, m: (m, n)),
        out_shape=jax.ShapeDtypeStruct((M, N), F32),
    )(*args)


def _ropehash_body(q_ref, k_ref, cos_ref, sin_ref, p_ref, b_ref,
                   qr_ref, kr_ref, qh_ref, kh_ref):
    cos = cos_ref[...]
    sin = sin_ref[...]
    p = p_ref[0]
    b = b_ref[0]
    hd = cos.shape[1]

    def rope(x):
        x1 = x[:, :hd // 2]
        x2 = x[:, hd // 2:]
        rot = jnp.concatenate([-x2, x1], axis=1)
        return x * cos + rot * sin

    qr = rope(q_ref[...]).astype(BF16)
    kr = rope(k_ref[...]).astype(BF16)
    qr_ref[...] = qr
    kr_ref[...] = kr
    pb = p.astype(BF16)
    qh_ref[...] = jnp.sign(_dot(qr, pb) + b).astype(BF16)
    kh_ref[...] = jnp.sign(_dot(kr, pb) + b).astype(BF16)


def _ropehash_p(qkv, cos, sin, proj, bias, S, D, H, KVH, HD):
    rep = H // KVH
    out_sd = jax.ShapeDtypeStruct((S, H * HD), BF16)
    return pl.pallas_call(
        _ropehash_body,
        grid=(H,),
        in_specs=[
            pl.BlockSpec((S, HD), lambda h: (0, h)),
            pl.BlockSpec((S, HD), lambda h: (0, H + h // rep)),
            pl.BlockSpec((S, HD), lambda h: (0, 0)),
            pl.BlockSpec((S, HD), lambda h: (0, 0)),
            pl.BlockSpec((1, HD, HD), lambda h: (h, 0, 0)),
            pl.BlockSpec((1, 1, HD), lambda h: (h, 0, 0)),
        ],
        out_specs=[pl.BlockSpec((S, HD), lambda h: (0, h)) for _ in range(4)],
        out_shape=[out_sd, out_sd, out_sd, out_sd],
    )(qkv, qkv, cos, sin, proj, bias)


def _attn_body(qh_ref, kh_ref, qr_ref, kr_ref, v_ref, u_ref, o_ref, *,
               qb, s, hd, k_keep, iters, ch, q0):
    draft = _dot(qh_ref[...], kh_ref[...], trans_b=True)
    row = q0 + lax.broadcasted_iota(jnp.int32, (qb, s), 0)
    col = lax.broadcasted_iota(jnp.int32, (qb, s), 1)
    inband = col <= row
    draft_bf = jnp.where(inband, draft, -1000.0).astype(BF16)

    kf = jnp.float32(k_keep)
    lo = jnp.full((qb, 1), -float(hd), F32)
    hi = jnp.full((qb, 1), float(hd), F32)
    cnt_hi = jnp.zeros((qb, 1), F32)
    for _ in range(iters):
        mid = jnp.floor((lo + hi) * 0.5)
        cnt = jnp.sum((draft_bf >= mid.astype(BF16)).astype(F32),
                      axis=1, keepdims=True)
        ge = cnt >= kf
        lo = jnp.where(ge, mid, lo)
        hi = jnp.where(ge, hi, mid)
        cnt_hi = jnp.where(ge, cnt_hi, cnt)
    t_bf = lo.astype(BF16)
    need = kf - cnt_hi

    tie = draft_bf == t_bf
    tie_bf = tie.astype(BF16)
    u = u_ref[...]
    ranks = []
    offset = jnp.zeros((qb, 1), F32)
    for c in range(s // ch):
        chunk = lax.slice(tie_bf, (0, c * ch), (qb, (c + 1) * ch))
        wc = _dot(chunk, u)
        ranks.append(wc + offset)
        offset = offset + lax.slice(wc, (0, ch - 1), (qb, ch))
    rank = jnp.concatenate(ranks, axis=1)
    short = (row < k_keep) & inband
    keep = (draft_bf > t_bf) | (tie & (rank <= need)) | short

    sc = _dot(qr_ref[...], kr_ref[...], trans_b=True) * (1.0 / math.sqrt(hd))
    sc = jnp.where(keep, sc, -1e30)
    m = jnp.max(sc, axis=1, keepdims=True)
    p = jnp.exp(sc - m)
    probs = (p / jnp.sum(p, axis=1, keepdims=True)).astype(BF16)
    o_ref[...] = _dot(probs, v_ref[...].astype(BF16)).astype(BF16)


def _attn_p(qh, kh, qr, kr, qkv, tri, S, D, H, KVH, HD, k_keep):
    qb = min(512, S)
    ch = min(256, S)
    iters = max(1, math.ceil(math.log2(2 * hd_range(HD))))
    rep = H // KVH
    outs = []
    for i in range(S // qb):
        s_eff = (i + 1) * qb
        body = functools.partial(_attn_body, qb=qb, s=s_eff, hd=HD,
                                 k_keep=k_keep, iters=iters, ch=ch, q0=i * qb)
        outs.append(pl.pallas_call(
            body,
            grid=(H,),
            in_specs=[
                pl.BlockSpec((qb, HD), lambda h, _i=i: (_i, h)),
                pl.BlockSpec((s_eff, HD), lambda h: (0, h)),
                pl.BlockSpec((qb, HD), lambda h, _i=i: (_i, h)),
                pl.BlockSpec((s_eff, HD), lambda h: (0, h)),
                pl.BlockSpec((s_eff, HD), lambda h: (0, H + KVH + h // rep)),
                pl.BlockSpec((ch, ch), lambda h: (0, 0)),
            ],
            out_specs=pl.BlockSpec((qb, HD), lambda h: (0, h)),
            out_shape=jax.ShapeDtypeStruct((qb, H * HD), BF16),
        )(qh, kh, qr, kr, qkv, tri))
    return jnp.concatenate(outs, axis=0) if len(outs) > 1 else outs[0]


def hd_range(HD):
    return HD


def _gate_up_body(h2_ref, wg_ref, wu_ref, act_ref):
    h2 = h2_ref[...]
    g = _dot(h2, wg_ref[...].astype(BF16))
    uu = _dot(h2, wu_ref[...].astype(BF16))
    act_ref[...] = (g * lax.logistic(g) * uu).astype(BF16)


def _gate_up_p(h2, wg, wu):
    S, D = h2.shape
    FF = wg.shape[1]
    fb = min(256, FF)
    return pl.pallas_call(
        _gate_up_body,
        grid=(FF // fb,),
        in_specs=[
            pl.BlockSpec((S, D), lambda f: (0, 0)),
            pl.BlockSpec((D, fb), lambda f: (0, f)),
            pl.BlockSpec((D, fb), lambda f: (0, f)),
        ],
        out_specs=pl.BlockSpec((S, fb), lambda f: (0, f)),
        out_shape=jax.ShapeDtypeStruct((S, FF), BF16),
    )(h2, wg, wu)


def kernel(hidden_states, Wq, Wk, Wv, Wo, hash_proj, hash_bias,
           ln1_w, ln2_w, Wgate, Wup, Wdown):
    B, S, D = hidden_states.shape
    HHD = Wq.shape[1]
    HD = hash_proj.shape[-1]
    H = HHD // HD
    KVH = Wk.shape[1] // HD
    theta = 10000.0
    num_remain = S - int(S * 0.9)
    num_remain = max(min(S, 128), num_remain)

    hs2 = hidden_states.reshape(S, D)

    inv_freq = 1.0 / (theta ** (jnp.arange(0, HD, 2, dtype=F32) / HD))
    tpos = jnp.arange(S, dtype=F32)
    freqs = jnp.outer(tpos, inv_freq)
    emb = jnp.concatenate([freqs, freqs], axis=-1)
    cos, sin = jnp.cos(emb), jnp.sin(emb)

    tri = (lax.broadcasted_iota(jnp.int32, (min(256, S),) * 2, 0)
           <= lax.broadcasted_iota(jnp.int32, (min(256, S),) * 2, 1)
           ).astype(BF16)

    h1 = _rmsnorm_p(hs2, ln1_w, odtype=BF16)
    wqkv = jnp.concatenate([Wq, Wk, Wv], axis=1).astype(BF16)
    qkv = _matmul_p(h1, wqkv, mb=512, nb=1024)

    proj = hash_proj.reshape(H, HD, HD)
    bias = hash_bias.reshape(H, 1, HD)
    qr, kr, qh, kh = _ropehash_p(qkv, cos, sin, proj, bias, S, D, H, KVH, HD)

    attn = _attn_p(qh, kh, qr, kr, qkv, tri, S, D, H, KVH, HD, num_remain)

    resid2 = _matmul_p(attn, Wo.astype(BF16), mb=512, nb=1024, resid=hs2)

    h2 = _rmsnorm_p(resid2, ln2_w, odtype=BF16)
    act = _gate_up_p(h2, Wgate, Wup)
    mlp = _matmul_p(act, Wdown.astype(BF16), mb=512, nb=512)
    return (resid2 + mlp).reshape(B, S, D)


# speedup vs baseline: 48.1575x; 1.1757x over previous
import functools
import math

import jax
import jax.numpy as jnp
from jax import lax
from jax.experimental import pallas as pl
from jax.experimental.pallas import tpu as pltpu

F32 = jnp.float32
BF16 = jnp.bfloat16
HI = lax.Precision.HIGHEST


def _dot(a, b, precision=None, trans_b=False):
    dn = (((1,), (1 if trans_b else 0,)), ((), ()))
    return lax.dot_general(a, b, dn, precision=precision,
                           preferred_element_type=F32)


def _rms_body(x_ref, w_ref, o_ref, *, odtype):
    x = x_ref[...]
    var = jnp.mean(x * x, axis=1, keepdims=True)
    o_ref[...] = (x * lax.rsqrt(var + 1e-6) * w_ref[...]).astype(odtype)


def _rmsnorm_p(x, w, odtype=F32):
    S, D = x.shape
    mb = min(256, S)
    return pl.pallas_call(
        functools.partial(_rms_body, odtype=odtype),
        grid=(S // mb,),
        in_specs=[pl.BlockSpec((mb, D), lambda i: (i, 0)),
                  pl.BlockSpec((1, D), lambda i: (0, 0))],
        out_specs=pl.BlockSpec((mb, D), lambda i: (i, 0)),
        out_shape=jax.ShapeDtypeStruct((S, D), odtype),
    )(x, w.reshape(1, D))


def _mm_body(x_ref, w_ref, o_ref, *, prec):
    o_ref[...] = _dot(x_ref[...], w_ref[...], precision=prec)


def _mmres_body(x_ref, w_ref, r_ref, o_ref, *, prec):
    o_ref[...] = _dot(x_ref[...], w_ref[...], precision=prec) + r_ref[...]


def _matmul_p(x, w, mb, nb, prec=None, resid=None):
    M, K = x.shape
    _, N = w.shape
    mb, nb = min(mb, M), min(nb, N)
    grid = (N // nb, M // mb)
    in_specs = [pl.BlockSpec((mb, K), lambda n, m: (m, 0)),
                pl.BlockSpec((K, nb), lambda n, m: (0, n))]
    args = [x, w]
    if resid is not None:
        in_specs.append(pl.BlockSpec((mb, nb), lambda n, m: (m, n)))
        args.append(resid)
        body = functools.partial(_mmres_body, prec=prec)
    else:
        body = functools.partial(_mm_body, prec=prec)
    return pl.pallas_call(
        body, grid=grid,
        in_specs=in_specs,
        out_specs=pl.BlockSpec((mb, nb), lambda n, m: (m, n)),
        out_shape=jax.ShapeDtypeStruct((M, N), F32),
    )(*args)


def _ropehash_body(q_ref, k_ref, cos_ref, sin_ref, p_ref, b_ref,
                   qr_ref, kr_ref, qh_ref, kh_ref):
    cos = cos_ref[...]
    sin = sin_ref[...]
    p = p_ref[0]
    b = b_ref[0]
    hd = cos.shape[1]

    def rope(x):
        x1 = x[:, :hd // 2]
        x2 = x[:, hd // 2:]
        rot = jnp.concatenate([-x2, x1], axis=1)
        return x * cos + rot * sin

    qr = rope(q_ref[...]).astype(BF16)
    kr = rope(k_ref[...]).astype(BF16)
    qr_ref[...] = qr
    kr_ref[...] = kr
    pb = p.astype(BF16)
    qh_ref[...] = jnp.sign(_dot(qr, pb) + b).astype(BF16)
    kh_ref[...] = jnp.sign(_dot(kr, pb) + b).astype(BF16)


def _ropehash_p(qkv, cos, sin, proj, bias, S, D, H, KVH, HD):
    rep = H // KVH
    out_sd = jax.ShapeDtypeStruct((S, H * HD), BF16)
    return pl.pallas_call(
        _ropehash_body,
        grid=(H,),
        in_specs=[
            pl.BlockSpec((S, HD), lambda h: (0, h)),
            pl.BlockSpec((S, HD), lambda h: (0, H + h // rep)),
            pl.BlockSpec((S, HD), lambda h: (0, 0)),
            pl.BlockSpec((S, HD), lambda h: (0, 0)),
            pl.BlockSpec((1, HD, HD), lambda h: (h, 0, 0)),
            pl.BlockSpec((1, 1, HD), lambda h: (h, 0, 0)),
        ],
        out_specs=[pl.BlockSpec((S, HD), lambda h: (0, h)) for _ in range(4)],
        out_shape=[out_sd, out_sd, out_sd, out_sd],
    )(qkv, qkv, cos, sin, proj, bias)


def _attn_body(qh_ref, kh_ref, qr_ref, kr_ref, v_ref, u_ref, o_ref, *,
               qb, s, hd, k_keep, iters, ch, q0):
    draft = _dot(qh_ref[...], kh_ref[...], trans_b=True)
    row = q0 + lax.broadcasted_iota(jnp.int32, (qb, s), 0)
    col = lax.broadcasted_iota(jnp.int32, (qb, s), 1)
    inband = col <= row
    draftm = jnp.where(inband, draft, -1000.0)

    kf = jnp.float32(k_keep)
    lo = jnp.full((qb, 1), -float(hd), F32)
    hi = jnp.full((qb, 1), float(hd), F32)
    cnt_hi = jnp.zeros((qb, 1), F32)
    for _ in range(iters):
        mid = jnp.floor((lo + hi) * 0.5)
        cnt = jnp.sum((draftm >= mid).astype(F32), axis=1, keepdims=True)
        ge = cnt >= kf
        lo = jnp.where(ge, mid, lo)
        hi = jnp.where(ge, hi, mid)
        cnt_hi = jnp.where(ge, cnt_hi, cnt)
    t = lo
    need = kf - cnt_hi

    tie = draftm == t
    tie_bf = tie.astype(BF16)
    u = u_ref[...]
    ranks = []
    offset = jnp.zeros((qb, 1), F32)
    for c in range(s // ch):
        chunk = lax.slice(tie_bf, (0, c * ch), (qb, (c + 1) * ch))
        wc = _dot(chunk, u)
        ranks.append(wc + offset)
        offset = offset + lax.slice(wc, (0, ch - 1), (qb, ch))
    rank = jnp.concatenate(ranks, axis=1)
    short = (row < k_keep) & inband
    keep = (draftm > t) | (tie & (rank <= need)) | short

    sc = _dot(qr_ref[...], kr_ref[...], trans_b=True) * (1.0 / math.sqrt(hd))
    sc = jnp.where(keep, sc, -1e30)
    m = jnp.max(sc, axis=1, keepdims=True)
    p = jnp.exp(sc - m)
    probs = (p / jnp.sum(p, axis=1, keepdims=True)).astype(BF16)
    o_ref[...] = _dot(probs, v_ref[...].astype(BF16)).astype(BF16)


def _attn_p(qh, kh, qr, kr, qkv, tri, S, D, H, KVH, HD, k_keep):
    qb = min(256, S)
    ch = min(256, S)
    iters = max(1, math.ceil(math.log2(2 * hd_range(HD))))
    rep = H // KVH
    outs = []
    for i in range(S // qb):
        s_eff = (i + 1) * qb
        body = functools.partial(_attn_body, qb=qb, s=s_eff, hd=HD,
                                 k_keep=k_keep, iters=iters, ch=ch, q0=i * qb)
        outs.append(pl.pallas_call(
            body,
            grid=(H,),
            in_specs=[
                pl.BlockSpec((qb, HD), lambda h, _i=i: (_i, h)),
                pl.BlockSpec((s_eff, HD), lambda h: (0, h)),
                pl.BlockSpec((qb, HD), lambda h, _i=i: (_i, h)),
                pl.BlockSpec((s_eff, HD), lambda h: (0, h)),
                pl.BlockSpec((s_eff, HD), lambda h: (0, H + KVH + h // rep)),
                pl.BlockSpec((ch, ch), lambda h: (0, 0)),
            ],
            out_specs=pl.BlockSpec((qb, HD), lambda h: (0, h)),
            out_shape=jax.ShapeDtypeStruct((qb, H * HD), BF16),
        )(qh, kh, qr, kr, qkv, tri))
    return jnp.concatenate(outs, axis=0) if len(outs) > 1 else outs[0]


def hd_range(HD):
    return HD


def _gate_up_body(h2_ref, wg_ref, wu_ref, act_ref):
    h2 = h2_ref[...]
    g = _dot(h2, wg_ref[...].astype(BF16))
    uu = _dot(h2, wu_ref[...].astype(BF16))
    act_ref[...] = (g * lax.logistic(g) * uu).astype(BF16)


def _gate_up_p(h2, wg, wu):
    S, D = h2.shape
    FF = wg.shape[1]
    fb = min(256, FF)
    return pl.pallas_call(
        _gate_up_body,
        grid=(FF // fb,),
        in_specs=[
            pl.BlockSpec((S, D), lambda f: (0, 0)),
            pl.BlockSpec((D, fb), lambda f: (0, f)),
            pl.BlockSpec((D, fb), lambda f: (0, f)),
        ],
        out_specs=pl.BlockSpec((S, fb), lambda f: (0, f)),
        out_shape=jax.ShapeDtypeStruct((S, FF), BF16),
    )(h2, wg, wu)


def kernel(hidden_states, Wq, Wk, Wv, Wo, hash_proj, hash_bias,
           ln1_w, ln2_w, Wgate, Wup, Wdown):
    B, S, D = hidden_states.shape
    HHD = Wq.shape[1]
    HD = hash_proj.shape[-1]
    H = HHD // HD
    KVH = Wk.shape[1] // HD
    theta = 10000.0
    num_remain = S - int(S * 0.9)
    num_remain = max(min(S, 128), num_remain)

    hs2 = hidden_states.reshape(S, D)

    inv_freq = 1.0 / (theta ** (jnp.arange(0, HD, 2, dtype=F32) / HD))
    tpos = jnp.arange(S, dtype=F32)
    freqs = jnp.outer(tpos, inv_freq)
    emb = jnp.concatenate([freqs, freqs], axis=-1)
    cos, sin = jnp.cos(emb), jnp.sin(emb)

    tri = (lax.broadcasted_iota(jnp.int32, (min(256, S),) * 2, 0)
           <= lax.broadcasted_iota(jnp.int32, (min(256, S),) * 2, 1)
           ).astype(BF16)

    h1 = _rmsnorm_p(hs2, ln1_w, odtype=BF16)
    wqkv = jnp.concatenate([Wq, Wk, Wv], axis=1).astype(BF16)
    qkv = _matmul_p(h1, wqkv, mb=512, nb=1024)

    proj = hash_proj.reshape(H, HD, HD)
    bias = hash_bias.reshape(H, 1, HD)
    qr, kr, qh, kh = _ropehash_p(qkv, cos, sin, proj, bias, S, D, H, KVH, HD)

    attn = _attn_p(qh, kh, qr, kr, qkv, tri, S, D, H, KVH, HD, num_remain)

    resid2 = _matmul_p(attn, Wo.astype(BF16), mb=512, nb=1024, resid=hs2)

    h2 = _rmsnorm_p(resid2, ln2_w, odtype=BF16)
    act = _gate_up_p(h2, Wgate, Wup)
    mlp = _matmul_p(act, Wdown.astype(BF16), mb=512, nb=512)
    return (resid2 + mlp).reshape(B, S, D)


# speedup vs baseline: 49.0296x; 1.0181x over previous
import functools
import math

import jax
import jax.numpy as jnp
from jax import lax
from jax.experimental import pallas as pl
from jax.experimental.pallas import tpu as pltpu

F32 = jnp.float32
BF16 = jnp.bfloat16
HI = lax.Precision.HIGHEST


def _dot(a, b, precision=None, trans_b=False):
    dn = (((1,), (1 if trans_b else 0,)), ((), ()))
    return lax.dot_general(a, b, dn, precision=precision,
                           preferred_element_type=F32)


def _rms_body(x_ref, w_ref, o_ref, *, odtype):
    x = x_ref[...]
    var = jnp.mean(x * x, axis=1, keepdims=True)
    o_ref[...] = (x * lax.rsqrt(var + 1e-6) * w_ref[...]).astype(odtype)


def _rmsnorm_p(x, w, odtype=F32):
    S, D = x.shape
    mb = min(256, S)
    return pl.pallas_call(
        functools.partial(_rms_body, odtype=odtype),
        grid=(S // mb,),
        in_specs=[pl.BlockSpec((mb, D), lambda i: (i, 0)),
                  pl.BlockSpec((1, D), lambda i: (0, 0))],
        out_specs=pl.BlockSpec((mb, D), lambda i: (i, 0)),
        out_shape=jax.ShapeDtypeStruct((S, D), odtype),
    )(x, w.reshape(1, D))


def _mm_body(x_ref, w_ref, o_ref, *, prec):
    o_ref[...] = _dot(x_ref[...], w_ref[...], precision=prec)


def _mmres_body(x_ref, w_ref, r_ref, o_ref, *, prec):
    o_ref[...] = _dot(x_ref[...], w_ref[...], precision=prec) + r_ref[...]


def _matmul_p(x, w, mb, nb, prec=None, resid=None):
    M, K = x.shape
    _, N = w.shape
    mb, nb = min(mb, M), min(nb, N)
    grid = (N // nb, M // mb)
    in_specs = [pl.BlockSpec((mb, K), lambda n, m: (m, 0)),
                pl.BlockSpec((K, nb), lambda n, m: (0, n))]
    args = [x, w]
    if resid is not None:
        in_specs.append(pl.BlockSpec((mb, nb), lambda n, m: (m, n)))
        args.append(resid)
        body = functools.partial(_mmres_body, prec=prec)
    else:
        body = functools.partial(_mm_body, prec=prec)
    return pl.pallas_call(
        body, grid=grid,
        in_specs=in_specs,
        out_specs=pl.BlockSpec((mb, nb), lambda n, m: (m, n)),
        out_shape=jax.ShapeDtypeStruct((M, N), F32),
    )(*args)


def _ropehash_body(q_ref, k_ref, cos_ref, sin_ref, p_ref, b_ref,
                   qr_ref, kr_ref, qh_ref, kh_ref):
    cos = cos_ref[...]
    sin = sin_ref[...]
    p = p_ref[0]
    b = b_ref[0]
    hd = cos.shape[1]

    def rope(x):
        x1 = x[:, :hd // 2]
        x2 = x[:, hd // 2:]
        rot = jnp.concatenate([-x2, x1], axis=1)
        return x * cos + rot * sin

    qr = rope(q_ref[...]).astype(BF16)
    kr = rope(k_ref[...]).astype(BF16)
    qr_ref[...] = qr
    kr_ref[...] = kr
    pb = p.astype(BF16)
    qh_ref[...] = jnp.sign(_dot(qr, pb) + b).astype(BF16)
    kh_ref[...] = jnp.sign(_dot(kr, pb) + b).astype(BF16)


def _ropehash_p(qkv, cos, sin, proj, bias, S, D, H, KVH, HD):
    rep = H // KVH
    out_sd = jax.ShapeDtypeStruct((S, H * HD), BF16)
    return pl.pallas_call(
        _ropehash_body,
        grid=(H,),
        in_specs=[
            pl.BlockSpec((S, HD), lambda h: (0, h)),
            pl.BlockSpec((S, HD), lambda h: (0, H + h // rep)),
            pl.BlockSpec((S, HD), lambda h: (0, 0)),
            pl.BlockSpec((S, HD), lambda h: (0, 0)),
            pl.BlockSpec((1, HD, HD), lambda h: (h, 0, 0)),
            pl.BlockSpec((1, 1, HD), lambda h: (h, 0, 0)),
        ],
        out_specs=[pl.BlockSpec((S, HD), lambda h: (0, h)) for _ in range(4)],
        out_shape=[out_sd, out_sd, out_sd, out_sd],
    )(qkv, qkv, cos, sin, proj, bias)


def _attn_body(qh_ref, kh_ref, qr_ref, kr_ref, v_ref, u_ref, o_ref, *,
               qb, s, hd, k_keep, iters, ch, q0):
    draft = _dot(qh_ref[...], kh_ref[...], trans_b=True)
    row = q0 + lax.broadcasted_iota(jnp.int32, (qb, s), 0)
    col = lax.broadcasted_iota(jnp.int32, (qb, s), 1)
    inband = col <= row
    draftm = jnp.where(inband, draft, -1000.0)

    kf = jnp.float32(k_keep)
    lo = jnp.full((qb, 1), -float(hd), F32)
    hi = jnp.full((qb, 1), float(hd), F32)
    cnt_hi = jnp.zeros((qb, 1), F32)
    for _ in range(iters):
        mid = jnp.floor((lo + hi) * 0.5)
        cnt = jnp.sum((draftm >= mid).astype(F32), axis=1, keepdims=True)
        ge = cnt >= kf
        lo = jnp.where(ge, mid, lo)
        hi = jnp.where(ge, hi, mid)
        cnt_hi = jnp.where(ge, cnt_hi, cnt)
    t = lo
    need = kf - cnt_hi

    tie = draftm == t
    tie_bf = tie.astype(BF16)
    u = u_ref[...]
    ranks = []
    offset = jnp.zeros((qb, 1), F32)
    for c in range(s // ch):
        chunk = lax.slice(tie_bf, (0, c * ch), (qb, (c + 1) * ch))
        wc = _dot(chunk, u)
        ranks.append(wc + offset)
        offset = offset + lax.slice(wc, (0, ch - 1), (qb, ch))
    rank = jnp.concatenate(ranks, axis=1)
    short = (row < k_keep) & inband
    keep = (draftm > t) | (tie & (rank <= need)) | short

    sc = _dot(qr_ref[...], kr_ref[...], trans_b=True) * (1.0 / math.sqrt(hd))
    sc = jnp.where(keep, sc, -1e30)
    m = jnp.max(sc, axis=1, keepdims=True)
    p = jnp.exp(sc - m)
    probs = (p / jnp.sum(p, axis=1, keepdims=True)).astype(BF16)
    o_ref[...] = _dot(probs, v_ref[...].astype(BF16)).astype(BF16)


def _attn_p(qh, kh, qr, kr, qkv, tri, S, D, H, KVH, HD, k_keep):
    qb = min(256, S)
    ch = min(256, S)
    iters = max(1, math.ceil(math.log2(2 * hd_range(HD))))
    rep = H // KVH
    outs = []
    for i in range(S // qb):
        s_eff = (i + 1) * qb
        body = functools.partial(_attn_body, qb=qb, s=s_eff, hd=HD,
                                 k_keep=k_keep, iters=iters, ch=ch, q0=i * qb)
        outs.append(pl.pallas_call(
            body,
            grid=(H,),
            in_specs=[
                pl.BlockSpec((qb, HD), lambda h, _i=i: (_i, h)),
                pl.BlockSpec((s_eff, HD), lambda h: (0, h)),
                pl.BlockSpec((qb, HD), lambda h, _i=i: (_i, h)),
                pl.BlockSpec((s_eff, HD), lambda h: (0, h)),
                pl.BlockSpec((s_eff, HD), lambda h: (0, H + KVH + h // rep)),
                pl.BlockSpec((ch, ch), lambda h: (0, 0)),
            ],
            out_specs=pl.BlockSpec((qb, HD), lambda h: (0, h)),
            out_shape=jax.ShapeDtypeStruct((qb, H * HD), BF16),
        )(qh, kh, qr, kr, qkv, tri))
    return jnp.concatenate(outs, axis=0) if len(outs) > 1 else outs[0]


def hd_range(HD):
    return HD


def _gate_up_body(h2_ref, wg_ref, wu_ref, act_ref):
    h2 = h2_ref[...]
    g = _dot(h2, wg_ref[...].astype(BF16))
    uu = _dot(h2, wu_ref[...].astype(BF16))
    act_ref[...] = (g * lax.logistic(g) * uu).astype(BF16)


def _gate_up_p(h2, wg, wu):
    S, D = h2.shape
    FF = wg.shape[1]
    fb = min(256, FF)
    return pl.pallas_call(
        _gate_up_body,
        grid=(FF // fb,),
        in_specs=[
            pl.BlockSpec((S, D), lambda f: (0, 0)),
            pl.BlockSpec((D, fb), lambda f: (0, f)),
            pl.BlockSpec((D, fb), lambda f: (0, f)),
        ],
        out_specs=pl.BlockSpec((S, fb), lambda f: (0, f)),
        out_shape=jax.ShapeDtypeStruct((S, FF), BF16),
    )(h2, wg, wu)


def kernel(hidden_states, Wq, Wk, Wv, Wo, hash_proj, hash_bias,
           ln1_w, ln2_w, Wgate, Wup, Wdown):
    B, S, D = hidden_states.shape
    HHD = Wq.shape[1]
    HD = hash_proj.shape[-1]
    H = HHD // HD
    KVH = Wk.shape[1] // HD
    theta = 10000.0
    num_remain = S - int(S * 0.9)
    num_remain = max(min(S, 128), num_remain)

    hs2 = hidden_states.reshape(S, D)

    inv_freq = 1.0 / (theta ** (jnp.arange(0, HD, 2, dtype=F32) / HD))
    tpos = jnp.arange(S, dtype=F32)
    freqs = jnp.outer(tpos, inv_freq)
    emb = jnp.concatenate([freqs, freqs], axis=-1)
    cos, sin = jnp.cos(emb), jnp.sin(emb)

    tri = (lax.broadcasted_iota(jnp.int32, (min(256, S),) * 2, 0)
           <= lax.broadcasted_iota(jnp.int32, (min(256, S),) * 2, 1)
           ).astype(BF16)

    h1 = _rmsnorm_p(hs2, ln1_w, odtype=BF16)
    wqkv = jnp.concatenate([Wq.astype(BF16), Wk.astype(BF16),
                            Wv.astype(BF16)], axis=1)
    qkv = _matmul_p(h1, wqkv, mb=512, nb=1024)

    proj = hash_proj.reshape(H, HD, HD)
    bias = hash_bias.reshape(H, 1, HD)
    qr, kr, qh, kh = _ropehash_p(qkv, cos, sin, proj, bias, S, D, H, KVH, HD)

    attn = _attn_p(qh, kh, qr, kr, qkv, tri, S, D, H, KVH, HD, num_remain)

    resid2 = _matmul_p(attn, Wo.astype(BF16), mb=512, nb=1024, resid=hs2)

    h2 = _rmsnorm_p(resid2, ln2_w, odtype=BF16)
    act = _gate_up_p(h2, Wgate, Wup)
    out = _matmul_p(act, Wdown.astype(BF16), mb=512, nb=512, resid=resid2)
    return out.reshape(B, S, D)
